# baseline (speedup 1.0000x reference)
import jax
import jax.numpy as jnp
from jax.experimental import pallas as pl
from jax.experimental.pallas import tpu as pltpu

_VOCAB = 8192
_D = 128
_G = 512
_STEPS = 64
_CHUNK = 2048
_HI = jax.lax.Precision.HIGHEST


def _decoder_kernel(x0_ref, inp_ref, S_ref, Wi_ref, b_ref, ew3_ref,
                    res_ref, dec_ref, iwn_ref):
    bconst = (jnp.dot(inp_ref[...], Wi_ref[...],
                      preferred_element_type=jnp.float32, precision=_HI)
              + b_ref[...])
    S = S_ref[...]

    def gates_to_state(t, c):
        ti = t[:, 0:128]
        tf = t[:, 128:256]
        to = t[:, 256:384]
        tg = t[:, 384:512]
        c_new = 0.5 * ((tf * c + c) + (ti * tg + tg))
        h_new = (0.5 * to + 0.5) * jnp.tanh(c_new)
        return h_new, c_new

    zero = jnp.zeros((1, _D), jnp.float32)
    h, c = zero, x0_ref[...]
    for step in range(_STEPS):
        c_hi = c.astype(jnp.bfloat16)
        c_lo = (c - c_hi.astype(jnp.float32)).astype(jnp.bfloat16)
        h_hi = h.astype(jnp.bfloat16)
        h_lo = (h - h_hi.astype(jnp.float32)).astype(jnp.bfloat16)
        z = jnp.concatenate(
            [jnp.concatenate([c_hi, h_hi], axis=1),
             jnp.concatenate([c_lo, h_lo], axis=1)], axis=0)
        out = jnp.dot(z, S, preferred_element_type=jnp.float32)
        gates = (out[0:1, 0:512] + out[0:1, 512:1024] + out[1:2, 0:512]
                 + bconst)
        t = jnp.tanh(gates)
        h, c = gates_to_state(t, zero if step == 0 else c)
        res_ref[step:step + 1, :] = c

    w_f32 = (ew3_ref[0:_D, :].astype(jnp.float32)
             + ew3_ref[_D:2 * _D, :].astype(jnp.float32))
    iwn_ref[...] = 1.0 / jnp.sqrt(jnp.sum(w_f32 * w_f32, axis=0,
                                          keepdims=True))

    res = res_ref[...]
    r_hi = res.astype(jnp.bfloat16)
    r_lo = (res - r_hi.astype(jnp.float32)).astype(jnp.bfloat16)
    r3 = jnp.concatenate([r_hi, r_hi, r_lo], axis=1)

    best_val = jnp.full((_STEPS, 1), -jnp.inf, jnp.float32)
    best_idx = jnp.zeros((_STEPS, 1), jnp.int32)
    for k in range(_VOCAB // _CHUNK):
        off = k * _CHUNK
        sims = (jnp.dot(r3, ew3_ref[:, off:off + _CHUNK],
                        preferred_element_type=jnp.float32)
                * iwn_ref[:, off:off + _CHUNK])
        cmax = jnp.max(sims, axis=1, keepdims=True)
        gidx = jax.lax.broadcasted_iota(jnp.int32, (_STEPS, _CHUNK), 1) + off
        cidx = jnp.min(jnp.where(sims == cmax, gidx, jnp.int32(2**31 - 1)),
                       axis=1, keepdims=True)
        take = cmax > best_val
        best_val = jnp.where(take, cmax, best_val)
        best_idx = jnp.where(take, cidx, best_idx)
    dec_ref[...] = best_idx


def _rearrange(w):
    return jnp.concatenate(
        [0.5 * w[0:_D], 0.5 * w[_D:2 * _D], 0.5 * w[3 * _D:4 * _D],
         w[2 * _D:3 * _D]], axis=0)


def _hi_lo(w):
    hi = w.astype(jnp.bfloat16)
    lo = (w - hi.astype(jnp.float32)).astype(jnp.bfloat16)
    return hi, lo


def kernel(inp, embed_weight, W_ih, W_hh, b_ih, b_hh):
    x0 = embed_weight[0:1, :]
    inp_row = inp.reshape(1, _D)
    Wx = _rearrange(W_ih[:, :_D]).T
    Wh = _rearrange(W_hh).T
    Wi = _rearrange(W_ih[:, _D:]).T
    b = _rearrange((b_ih + b_hh).reshape(_G, 1)).reshape(1, _G)
    xh, xl = _hi_lo(Wx)
    hh, hl = _hi_lo(Wh)
    S = jnp.concatenate(
        [jnp.concatenate([xh, hh], axis=0),
         jnp.concatenate([xl, hl], axis=0)], axis=1)
    eh, el = _hi_lo(embed_weight)
    ew3 = jnp.concatenate([eh, el, eh], axis=1).T

    res, dec = pl.pallas_call(
        _decoder_kernel,
        out_shape=[
            jax.ShapeDtypeStruct((_STEPS, _D), jnp.float32),
            jax.ShapeDtypeStruct((_STEPS, 1), jnp.int32),
        ],
        scratch_shapes=[pltpu.VMEM((1, _VOCAB), jnp.float32)],
    )(x0, inp_row, S, Wi, b, ew3)
    return res, dec.reshape(_STEPS)


# speedup vs baseline: 1.2890x; 1.2890x over previous
import jax
import jax.numpy as jnp
from jax.experimental import pallas as pl
from jax.experimental.pallas import tpu as pltpu

_VOCAB = 8192
_D = 128
_G = 512
_STEPS = 64
_CHUNK = 2048
_HI = jax.lax.Precision.HIGHEST


def _decoder_kernel(x0_ref, inp_ref, S_ref, Wi_ref, b_ref, ewt_ref,
                    res_ref, dec_ref, iwn_ref, ew2_ref):
    bconst = (jnp.dot(inp_ref[...], Wi_ref[...],
                      preferred_element_type=jnp.float32, precision=_HI)
              + b_ref[...])
    S = S_ref[...]

    def gates_to_state(t, c):
        ti = t[:, 0:128]
        tf = t[:, 128:256]
        to = t[:, 256:384]
        tg = t[:, 384:512]
        c_new = 0.5 * ((tf * c + c) + (ti * tg + tg))
        h_new = (0.5 * to + 0.5) * jnp.tanh(c_new)
        return h_new, c_new

    zero = jnp.zeros((1, _D), jnp.float32)
    h, c = zero, x0_ref[...]
    for step in range(_STEPS):
        c_hi = c.astype(jnp.bfloat16)
        c_lo = (c - c_hi.astype(jnp.float32)).astype(jnp.bfloat16)
        h_hi = h.astype(jnp.bfloat16)
        h_lo = (h - h_hi.astype(jnp.float32)).astype(jnp.bfloat16)
        z = jnp.concatenate(
            [jnp.concatenate([c_hi, h_hi], axis=1),
             jnp.concatenate([c_lo, h_lo], axis=1)], axis=0)
        out = jnp.dot(z, S, preferred_element_type=jnp.float32)
        gates = (out[0:1, 0:512] + out[0:1, 512:1024] + out[1:2, 0:512]
                 + bconst)
        t = jnp.tanh(gates)
        h, c = gates_to_state(t, zero if step == 0 else c)
        res_ref[step:step + 1, :] = c

    ew_t = ewt_ref[...]
    t_hi = ew_t.astype(jnp.bfloat16)
    t_lo = (ew_t - t_hi.astype(jnp.float32)).astype(jnp.bfloat16)
    ew2_ref[...] = jnp.concatenate([t_hi, t_lo], axis=0)
    iwn_ref[...] = 1.0 / jnp.sqrt(jnp.sum(ew_t * ew_t, axis=0,
                                          keepdims=True))

    res = res_ref[...]
    r_hi = res.astype(jnp.bfloat16)
    r_lo = (res - r_hi.astype(jnp.float32)).astype(jnp.bfloat16)
    zb = jnp.zeros_like(r_lo)
    z2 = jnp.concatenate(
        [jnp.concatenate([r_hi, r_hi], axis=1),
         jnp.concatenate([r_lo, zb], axis=1)], axis=0)

    best_val = jnp.full((_STEPS, 1), -jnp.inf, jnp.float32)
    best_idx = jnp.zeros((_STEPS, 1), jnp.int32)
    for k in range(_VOCAB // _CHUNK):
        off = k * _CHUNK
        out = jnp.dot(z2, ew2_ref[:, off:off + _CHUNK],
                      preferred_element_type=jnp.float32)
        sims = ((out[0:_STEPS, :] + out[_STEPS:2 * _STEPS, :])
                * iwn_ref[:, off:off + _CHUNK])
        cmax = jnp.max(sims, axis=1, keepdims=True)
        gidx = jax.lax.broadcasted_iota(jnp.int32, (_STEPS, _CHUNK), 1) + off
        cidx = jnp.min(jnp.where(sims == cmax, gidx, jnp.int32(2**31 - 1)),
                       axis=1, keepdims=True)
        take = cmax > best_val
        best_val = jnp.where(take, cmax, best_val)
        best_idx = jnp.where(take, cidx, best_idx)
    dec_ref[...] = best_idx


def _rearrange(w):
    return jnp.concatenate(
        [0.5 * w[0:_D], 0.5 * w[_D:2 * _D], 0.5 * w[3 * _D:4 * _D],
         w[2 * _D:3 * _D]], axis=0)


def _hi_lo(w):
    hi = w.astype(jnp.bfloat16)
    lo = (w - hi.astype(jnp.float32)).astype(jnp.bfloat16)
    return hi, lo


def kernel(inp, embed_weight, W_ih, W_hh, b_ih, b_hh):
    x0 = embed_weight[0:1, :]
    inp_row = inp.reshape(1, _D)
    Wx = _rearrange(W_ih[:, :_D]).T
    Wh = _rearrange(W_hh).T
    Wi = _rearrange(W_ih[:, _D:]).T
    b = _rearrange((b_ih + b_hh).reshape(_G, 1)).reshape(1, _G)
    xh, xl = _hi_lo(Wx)
    hh, hl = _hi_lo(Wh)
    S = jnp.concatenate(
        [jnp.concatenate([xh, hh], axis=0),
         jnp.concatenate([xl, hl], axis=0)], axis=1)
    ew_t = embed_weight.T

    res, dec = pl.pallas_call(
        _decoder_kernel,
        out_shape=[
            jax.ShapeDtypeStruct((_STEPS, _D), jnp.float32),
            jax.ShapeDtypeStruct((_STEPS, 1), jnp.int32),
        ],
        scratch_shapes=[pltpu.VMEM((1, _VOCAB), jnp.float32),
                        pltpu.VMEM((2 * _D, _VOCAB), jnp.bfloat16)],
    )(x0, inp_row, S, Wi, b, ew_t)
    return res, dec.reshape(_STEPS)


# speedup vs baseline: 1.3309x; 1.0325x over previous
import jax
import jax.numpy as jnp
from jax.experimental import pallas as pl
from jax.experimental.pallas import tpu as pltpu

_VOCAB = 8192
_D = 128
_G = 512
_STEPS = 64
_CHUNK = 2048
_HI = jax.lax.Precision.HIGHEST


def _decoder_kernel(x0_ref, inp_ref, S_ref, Wi_ref, b_ref, ewt_ref,
                    res_ref, dec_ref, iwn_ref, ew2_ref):
    sc = jnp.concatenate(
        [jnp.full((1, 2 * _D), 0.5, jnp.float32),
         jnp.ones((1, _D), jnp.float32),
         jnp.full((1, _D), 0.5, jnp.float32)], axis=1)
    bconst = (jnp.dot(inp_ref[...], Wi_ref[...],
                      preferred_element_type=jnp.float32, precision=_HI)
              + b_ref[...]) * sc
    S = S_ref[...]

    def gates_to_state(t, c):
        ti = t[:, 0:128]
        tf = t[:, 128:256]
        tg = t[:, 256:384]
        to = t[:, 384:512]
        c_new = 0.5 * ((tf * c + c) + (ti * tg + tg))
        h_new = (0.5 * to + 0.5) * jnp.tanh(c_new)
        return h_new, c_new

    zero = jnp.zeros((1, _D), jnp.float32)
    h, c = zero, x0_ref[...]
    for step in range(_STEPS):
        c_hi = c.astype(jnp.bfloat16)
        c_lo = (c - c_hi.astype(jnp.float32)).astype(jnp.bfloat16)
        h_hi = h.astype(jnp.bfloat16)
        h_lo = (h - h_hi.astype(jnp.float32)).astype(jnp.bfloat16)
        z = jnp.concatenate(
            [jnp.concatenate([c_hi, h_hi], axis=1),
             jnp.concatenate([c_lo, h_lo], axis=1)], axis=0)
        out = jnp.dot(z, S, preferred_element_type=jnp.float32)
        gates = ((out[0:1, 0:512] + out[0:1, 512:1024] + out[1:2, 0:512])
                 * sc + bconst)
        t = jnp.tanh(gates)
        h, c = gates_to_state(t, zero if step == 0 else c)
        res_ref[step:step + 1, :] = c

    ew_t = ewt_ref[...]
    t_hi = ew_t.astype(jnp.bfloat16)
    t_lo = (ew_t - t_hi.astype(jnp.float32)).astype(jnp.bfloat16)
    ew2_ref[...] = jnp.concatenate([t_hi, t_lo], axis=0)
    iwn_ref[...] = 1.0 / jnp.sqrt(jnp.sum(ew_t * ew_t, axis=0,
                                          keepdims=True))

    res = res_ref[...]
    r_hi = res.astype(jnp.bfloat16)
    r_lo = (res - r_hi.astype(jnp.float32)).astype(jnp.bfloat16)
    zb = jnp.zeros_like(r_lo)
    z2 = jnp.concatenate(
        [jnp.concatenate([r_hi, r_hi], axis=1),
         jnp.concatenate([r_lo, zb], axis=1)], axis=0)

    best_val = jnp.full((_STEPS, 1), -jnp.inf, jnp.float32)
    best_idx = jnp.zeros((_STEPS, 1), jnp.int32)
    for k in range(_VOCAB // _CHUNK):
        off = k * _CHUNK
        out = jnp.dot(z2, ew2_ref[:, off:off + _CHUNK],
                      preferred_element_type=jnp.float32)
        sims = ((out[0:_STEPS, :] + out[_STEPS:2 * _STEPS, :])
                * iwn_ref[:, off:off + _CHUNK])
        cmax = jnp.max(sims, axis=1, keepdims=True)
        gidx = jax.lax.broadcasted_iota(jnp.int32, (_STEPS, _CHUNK), 1) + off
        cidx = jnp.min(jnp.where(sims == cmax, gidx, jnp.int32(2**31 - 1)),
                       axis=1, keepdims=True)
        take = cmax > best_val
        best_val = jnp.where(take, cmax, best_val)
        best_idx = jnp.where(take, cidx, best_idx)
    dec_ref[...] = best_idx


def kernel(inp, embed_weight, W_ih, W_hh, b_ih, b_hh):
    x0 = embed_weight[0:1, :]
    inp_row = inp.reshape(1, _D)
    wall_t = jnp.concatenate(
        [W_ih[:, :_D], W_hh, W_ih[:, _D:]], axis=1).T
    wxh = wall_t[0:2 * _D]
    hi = wxh.astype(jnp.bfloat16)
    lo = (wxh - hi.astype(jnp.float32)).astype(jnp.bfloat16)
    S = jnp.concatenate([hi, lo], axis=1)
    Wi = wall_t[2 * _D:3 * _D]
    b = (b_ih + b_hh).reshape(1, _G)
    ew_t = embed_weight.T

    res, dec = pl.pallas_call(
        _decoder_kernel,
        out_shape=[
            jax.ShapeDtypeStruct((_STEPS, _D), jnp.float32),
            jax.ShapeDtypeStruct((_STEPS, 1), jnp.int32),
        ],
        scratch_shapes=[pltpu.VMEM((1, _VOCAB), jnp.float32),
                        pltpu.VMEM((2 * _D, _VOCAB), jnp.bfloat16)],
    )(x0, inp_row, S, Wi, b, ew_t)
    return res, dec.reshape(_STEPS)
